# baseline (speedup 1.0000x reference)
import functools

import jax
import jax.numpy as jnp
from jax import lax
from jax.experimental import pallas as pl
from jax.experimental.pallas import tpu as pltpu
from jax.experimental.pallas import tpu_sc as plsc

VOCAB = 1000000
EMB = 64
PAD = 16
BATCH = 16384
SEQ = 200

PACK = 8
VP = VOCAB // PACK
NP = PAD * PACK

NC = 2
NS = 16
NW = NC * NS
ROWS_PER_W = BATCH // NW
CH = 8
NCHUNK = ROWS_PER_W // CH
TOK = CH * SEQ
GA, GB = 104, 96
G = 2 * CH

BV = 32768


def _proj_body(t_ref, w_ref, sel_ref, o_ref):
    out8 = lax.dot_general(t_ref[...].astype(jnp.bfloat16),
                           w_ref[...].astype(jnp.bfloat16),
                           dimension_numbers=(((0,), (0,)), ((), ())),
                           preferred_element_type=jnp.float32)
    out83 = out8.reshape(BV // PACK, PACK, NP)
    for s in range(PACK):
        o_ref[:, s * PAD:(s + 1) * PAD] = out83[:, s, s * PAD:(s + 1) * PAD]


def _project(table_t, wt_rep, sel):
    return pl.pallas_call(
        _proj_body,
        grid=((VOCAB + BV - 1) // BV,),
        compiler_params=pltpu.CompilerParams(
            dimension_semantics=("parallel",)),
        in_specs=[
            pl.BlockSpec((EMB, BV), lambda i: (0, i)),
            pl.BlockSpec((EMB, NP), lambda i: (0, 0)),
            pl.BlockSpec((PACK, NP), lambda i: (0, 0)),
        ],
        out_specs=pl.BlockSpec((BV // PACK, NP), lambda i: (i, 0)),
        out_shape=jax.ShapeDtypeStruct((VP, NP), jnp.float32),
    )(table_t, wt_rep, sel)


def _invb_body(l_ref, sel_ref, o_ref):
    o_ref[...] = lax.dot_general(1.0 / l_ref[...], sel_ref[...],
                                 dimension_numbers=(((1,), (0,)), ((), ())),
                                 preferred_element_type=jnp.float32)


def _inv_packed(lenf2d, sel):
    return pl.pallas_call(
        _invb_body,
        out_shape=jax.ShapeDtypeStruct((BATCH // PACK, NP), jnp.float32),
    )(lenf2d, sel)


def _sc_pool(proj, data, invb, bias16):
    mesh = plsc.VectorSubcoreMesh(core_axis_name="c", subcore_axis_name="s")

    @functools.partial(
        pl.kernel,
        mesh=mesh,
        compiler_params=pltpu.CompilerParams(use_tc_tiling_on_sc=False),
        out_type=jax.ShapeDtypeStruct((BATCH, PAD), jnp.float32),
        scratch_types=[
            pltpu.VMEM((2, CH, SEQ), jnp.int32),
            pltpu.VMEM((2, TOK, PAD), jnp.float32),
            pltpu.VMEM((CH, PAD), jnp.float32),
            pltpu.VMEM((NCHUNK, NP), jnp.float32),
            pltpu.VMEM((PAD,), jnp.float32),
            pltpu.SemaphoreType.DMA,
            pltpu.SemaphoreType.DMA,
        ],
    )
    def k(proj_hbm, data_hbm, invb_hbm, bias_hbm, out_hbm,
          idx_v, rows_v, out_v, inv_v, bias_v, sem0, sem1):
        cid = lax.axis_index("c")
        sid = lax.axis_index("s")
        wid = sid * NC + cid
        sems = (sem0, sem1)
        gbase = pl.multiple_of(wid * NCHUNK, CH)
        pltpu.sync_copy(invb_hbm.at[pl.ds(gbase, NCHUNK)], inv_v)
        pltpu.sync_copy(bias_hbm, bias_v)

        def gather_list(p):
            ib = idx_v.at[p]
            pairs = []
            for r in range(CH):
                pairs.append((proj_hbm.at[ib.at[r, pl.ds(0, GA)]],
                              rows_v.at[p].at[pl.ds(r * SEQ, GA)]))
                pairs.append((proj_hbm.at[ib.at[r, pl.ds(GA, GB)]],
                              rows_v.at[p].at[pl.ds(r * SEQ + GA, GB)]))
            return pairs

        def stage(c, p):
            base = pl.multiple_of(wid * ROWS_PER_W + c * CH, CH)
            pltpu.sync_copy(data_hbm.at[pl.ds(base, CH), :], idx_v.at[p])
            for src, dst in gather_list(p):
                pltpu.async_copy(src, dst, sems[p])

        def drain(p):
            for src, dst in gather_list(p):
                pltpu.make_async_copy(src, dst, sems[p]).wait()

        def compute(c, p):
            base = pl.multiple_of(wid * ROWS_PER_W + c * CH, CH)
            rbuf = rows_v.at[p]
            for r in range(CH):
                rb = r * SEQ

                def tok_body(t, accs):
                    a0, a1, a2, a3 = accs
                    q = rb + t * 4
                    return (a0 + rbuf[q, :], a1 + rbuf[q + 1, :],
                            a2 + rbuf[q + 2, :], a3 + rbuf[q + 3, :])

                z = jnp.zeros((PAD,), jnp.float32)
                a0, a1, a2, a3 = lax.fori_loop(0, SEQ // 4, tok_body,
                                               (z, z, z, z))
                iv = inv_v[c, pl.ds(r * PAD, PAD)]
                out_v[r, :] = ((a0 + a1) + (a2 + a3)) * iv + bias_v[...]
            pltpu.sync_copy(out_v, out_hbm.at[pl.ds(base, CH)])

        stage(0, 0)

        def pair_body(g, carry):
            stage(2 * g + 1, 1)
            drain(0)
            compute(2 * g, 0)

            @pl.when(g < NCHUNK // 2 - 1)
            def _():
                stage(2 * g + 2, 0)

            drain(1)
            compute(2 * g + 1, 1)
            return carry

        lax.fori_loop(0, NCHUNK // 2, pair_body, 0)

    return k(proj, data, invb, bias16)


def kernel(data, length, table, W, b):
    data = data.astype(jnp.int32)
    lenf2d = length.astype(jnp.float32).reshape(BATCH // PACK, PACK)
    wt = jnp.zeros((EMB, PAD), jnp.float32).at[:, :2].set(W.T)
    wt_rep = jnp.tile(wt, (1, PACK))
    sel = (jnp.arange(NP) // PAD == jnp.arange(PACK)[:, None]
           ).astype(jnp.float32)
    bias16 = jnp.zeros((PAD,), jnp.float32).at[:2].set(b)
    packed = _project(table.T, wt_rep, sel)
    proj = packed.reshape(VOCAB, PAD)
    invb = _inv_packed(lenf2d, sel)
    out = _sc_pool(proj, data, invb, bias16)
    return out[:, :2]


# speedup vs baseline: 1.1489x; 1.1489x over previous
import functools

import jax
import jax.numpy as jnp
from jax import lax
from jax.experimental import pallas as pl
from jax.experimental.pallas import tpu as pltpu
from jax.experimental.pallas import tpu_sc as plsc

VOCAB = 1000000
EMB = 64
PAD = 16
BATCH = 16384
SEQ = 200

PACK = 8
VP = VOCAB // PACK
NP = PAD * PACK

NC = 2
NS = 16
NW = NC * NS
ROWS_PER_W = BATCH // NW
CH = 8
NCHUNK = ROWS_PER_W // CH
TOK = CH * SEQ
GA, GB = 104, 96
G = 2 * CH

BV = 32768


def _proj_body(t_ref, w_ref, sel_ref, o_ref):
    out8 = lax.dot_general(t_ref[...], w_ref[...],
                           dimension_numbers=(((0,), (0,)), ((), ())),
                           preferred_element_type=jnp.float32)
    out83 = out8.reshape(BV // PACK, PACK, NP)
    for s in range(PACK):
        o_ref[:, s * PAD:(s + 1) * PAD] = out83[:, s, s * PAD:(s + 1) * PAD]


def _project(table_t, wt_rep, sel):
    return pl.pallas_call(
        _proj_body,
        grid=((VOCAB + BV - 1) // BV,),
        compiler_params=pltpu.CompilerParams(
            dimension_semantics=("parallel",)),
        in_specs=[
            pl.BlockSpec((EMB, BV), lambda i: (0, i)),
            pl.BlockSpec((EMB, NP), lambda i: (0, 0)),
            pl.BlockSpec((PACK, NP), lambda i: (0, 0)),
        ],
        out_specs=pl.BlockSpec((BV // PACK, NP), lambda i: (i, 0)),
        out_shape=jax.ShapeDtypeStruct((VP, NP), jnp.float32),
    )(table_t, wt_rep, sel)


def _invb_body(l_ref, sel_ref, o_ref):
    o_ref[...] = lax.dot_general(1.0 / l_ref[...], sel_ref[...],
                                 dimension_numbers=(((1,), (0,)), ((), ())),
                                 preferred_element_type=jnp.float32)


def _inv_packed(lenf2d, sel):
    return pl.pallas_call(
        _invb_body,
        out_shape=jax.ShapeDtypeStruct((BATCH // PACK, NP), jnp.float32),
    )(lenf2d, sel)


def _sc_pool(proj, data, invb, bias16):
    mesh = plsc.VectorSubcoreMesh(core_axis_name="c", subcore_axis_name="s")

    @functools.partial(
        pl.kernel,
        mesh=mesh,
        compiler_params=pltpu.CompilerParams(use_tc_tiling_on_sc=False),
        out_type=jax.ShapeDtypeStruct((BATCH, PAD), jnp.float32),
        scratch_types=[
            pltpu.VMEM((2, CH, SEQ), jnp.int32),
            pltpu.VMEM((2, TOK, PAD), jnp.float32),
            pltpu.VMEM((CH, PAD), jnp.float32),
            pltpu.VMEM((NCHUNK, NP), jnp.float32),
            pltpu.VMEM((PAD,), jnp.float32),
            pltpu.SemaphoreType.DMA,
            pltpu.SemaphoreType.DMA,
        ],
    )
    def k(proj_hbm, data_hbm, invb_hbm, bias_hbm, out_hbm,
          idx_v, rows_v, out_v, inv_v, bias_v, sem0, sem1):
        cid = lax.axis_index("c")
        sid = lax.axis_index("s")
        wid = sid * NC + cid
        sems = (sem0, sem1)
        gbase = pl.multiple_of(wid * NCHUNK, CH)
        pltpu.sync_copy(invb_hbm.at[pl.ds(gbase, NCHUNK)], inv_v)
        pltpu.sync_copy(bias_hbm, bias_v)

        def gather_list(p):
            ib = idx_v.at[p]
            pairs = []
            for r in range(CH):
                pairs.append((proj_hbm.at[ib.at[r, pl.ds(0, GA)]],
                              rows_v.at[p].at[pl.ds(r * SEQ, GA)]))
                pairs.append((proj_hbm.at[ib.at[r, pl.ds(GA, GB)]],
                              rows_v.at[p].at[pl.ds(r * SEQ + GA, GB)]))
            return pairs

        def stage(c, p):
            base = pl.multiple_of(wid * ROWS_PER_W + c * CH, CH)
            pltpu.sync_copy(data_hbm.at[pl.ds(base, CH), :], idx_v.at[p])
            for src, dst in gather_list(p):
                pltpu.async_copy(src, dst, sems[p])

        def drain(p):
            for src, dst in gather_list(p):
                pltpu.make_async_copy(src, dst, sems[p]).wait()

        def compute(c, p):
            base = pl.multiple_of(wid * ROWS_PER_W + c * CH, CH)
            rbuf = rows_v.at[p]
            for r in range(CH):
                rb = r * SEQ

                def tok_body(t, accs):
                    q = rb + t * 8
                    return tuple(a + rbuf[q + j, :]
                                 for j, a in enumerate(accs))

                z = jnp.zeros((PAD,), jnp.float32)
                accs = lax.fori_loop(0, SEQ // 8, tok_body, (z,) * 8)
                s0 = (accs[0] + accs[1]) + (accs[2] + accs[3])
                s1 = (accs[4] + accs[5]) + (accs[6] + accs[7])
                iv = inv_v[c, pl.ds(r * PAD, PAD)]
                out_v[r, :] = (s0 + s1) * iv + bias_v[...]
            pltpu.sync_copy(out_v, out_hbm.at[pl.ds(base, CH)])

        stage(0, 0)

        def pair_body(g, carry):
            stage(2 * g + 1, 1)
            drain(0)
            compute(2 * g, 0)

            @pl.when(g < NCHUNK // 2 - 1)
            def _():
                stage(2 * g + 2, 0)

            drain(1)
            compute(2 * g + 1, 1)
            return carry

        lax.fori_loop(0, NCHUNK // 2, pair_body, 0)

    return k(proj, data, invb, bias16)


def kernel(data, length, table, W, b):
    data = data.astype(jnp.int32)
    lenf2d = length.astype(jnp.float32).reshape(BATCH // PACK, PACK)
    wt = jnp.zeros((EMB, PAD), jnp.float32).at[:, :2].set(W.T)
    wt_rep = jnp.tile(wt, (1, PACK))
    sel = (jnp.arange(NP) // PAD == jnp.arange(PACK)[:, None]
           ).astype(jnp.float32)
    bias16 = jnp.zeros((PAD,), jnp.float32).at[:2].set(b)
    packed = _project(table.T, wt_rep, sel)
    proj = packed.reshape(VOCAB, PAD)
    invb = _inv_packed(lenf2d, sel)
    out = _sc_pool(proj, data, invb, bias16)
    return out[:, :2]


# speedup vs baseline: 1.1521x; 1.0027x over previous
import functools

import jax
import jax.numpy as jnp
from jax import lax
from jax.experimental import pallas as pl
from jax.experimental.pallas import tpu as pltpu
from jax.experimental.pallas import tpu_sc as plsc

VOCAB = 1000000
EMB = 64
PAD = 16
BATCH = 16384
SEQ = 200

PACK = 8
VP = VOCAB // PACK
NP = PAD * PACK

NC = 2
NS = 16
NW = NC * NS
ROWS_PER_W = BATCH // NW
CH = 8
NCHUNK = ROWS_PER_W // CH
TOK = CH * SEQ
GA, GB = 104, 96
G = 2 * CH

BV = 32768


def _proj_body(t_ref, w_ref, sel_ref, o_ref):
    out8 = lax.dot_general(t_ref[...], w_ref[...],
                           dimension_numbers=(((0,), (0,)), ((), ())),
                           preferred_element_type=jnp.float32)
    out83 = out8.reshape(BV // PACK, PACK, NP)
    for s in range(PACK):
        o_ref[:, s * PAD:(s + 1) * PAD] = out83[:, s, s * PAD:(s + 1) * PAD]


def _project(table_t, wt_rep, sel):
    return pl.pallas_call(
        _proj_body,
        grid=((VOCAB + BV - 1) // BV,),
        compiler_params=pltpu.CompilerParams(
            dimension_semantics=("parallel",)),
        in_specs=[
            pl.BlockSpec((EMB, BV), lambda i: (0, i)),
            pl.BlockSpec((EMB, NP), lambda i: (0, 0)),
            pl.BlockSpec((PACK, NP), lambda i: (0, 0)),
        ],
        out_specs=pl.BlockSpec((BV // PACK, NP), lambda i: (i, 0)),
        out_shape=jax.ShapeDtypeStruct((VP, NP), jnp.float32),
    )(table_t, wt_rep, sel)


def _invb_body(l_ref, sel_ref, o_ref):
    o_ref[...] = lax.dot_general(1.0 / l_ref[...], sel_ref[...],
                                 dimension_numbers=(((1,), (0,)), ((), ())),
                                 preferred_element_type=jnp.float32)


def _inv_packed(lenf2d, sel):
    return pl.pallas_call(
        _invb_body,
        out_shape=jax.ShapeDtypeStruct((BATCH // PACK, NP), jnp.float32),
    )(lenf2d, sel)


def _sc_pool(proj, data, invb, bias16):
    mesh = plsc.VectorSubcoreMesh(core_axis_name="c", subcore_axis_name="s")

    @functools.partial(
        pl.kernel,
        mesh=mesh,
        compiler_params=pltpu.CompilerParams(use_tc_tiling_on_sc=False),
        out_type=jax.ShapeDtypeStruct((BATCH, PAD), jnp.float32),
        scratch_types=[
            pltpu.VMEM((2, CH, SEQ), jnp.int32),
            pltpu.VMEM((2, TOK, PAD), jnp.float32),
            pltpu.VMEM((CH, PAD), jnp.float32),
            pltpu.VMEM((NCHUNK, NP), jnp.float32),
            pltpu.VMEM((PAD,), jnp.float32),
            pltpu.SemaphoreType.DMA,
            pltpu.SemaphoreType.DMA,
        ],
    )
    def k(proj_hbm, data_hbm, invb_hbm, bias_hbm, out_hbm,
          idx_v, rows_v, out_v, inv_v, bias_v, sem0, sem1):
        cid = lax.axis_index("c")
        sid = lax.axis_index("s")
        wid = sid * NC + cid
        sems = (sem0, sem1)
        gbase = pl.multiple_of(wid * NCHUNK, CH)
        pltpu.sync_copy(invb_hbm.at[pl.ds(gbase, NCHUNK)], inv_v)
        pltpu.sync_copy(bias_hbm, bias_v)

        def gather_list(p):
            ib = idx_v.at[p]
            pairs = []
            for r in range(CH):
                pairs.append((proj_hbm.at[ib.at[r, pl.ds(0, GA)]],
                              rows_v.at[p].at[pl.ds(r * SEQ, GA)]))
                pairs.append((proj_hbm.at[ib.at[r, pl.ds(GA, GB)]],
                              rows_v.at[p].at[pl.ds(r * SEQ + GA, GB)]))
            return pairs

        def stage(c, p):
            base = pl.multiple_of(wid * ROWS_PER_W + c * CH, CH)
            pltpu.sync_copy(data_hbm.at[pl.ds(base, CH), :], idx_v.at[p])
            for src, dst in gather_list(p):
                pltpu.async_copy(src, dst, sems[p])

        def drain(p):
            for src, dst in gather_list(p):
                pltpu.make_async_copy(src, dst, sems[p]).wait()

        def compute(c, p):
            base = pl.multiple_of(wid * ROWS_PER_W + c * CH, CH)
            rbuf = rows_v.at[p]
            for r in range(CH):
                rb = r * SEQ

                z = jnp.zeros((PAD,), jnp.float32)

                def tok_body(t, accs_in):
                    q = rb + t * 8
                    return tuple(a + rbuf[q + j, :]
                                 for j, a in enumerate(accs_in))

                accs = plsc.parallel_loop(0, SEQ // 8, unroll=2,
                                          carry=(z,) * 8)(tok_body)
                s0 = (accs[0] + accs[1]) + (accs[2] + accs[3])
                s1 = (accs[4] + accs[5]) + (accs[6] + accs[7])
                iv = inv_v[c, pl.ds(r * PAD, PAD)]
                out_v[r, :] = (s0 + s1) * iv + bias_v[...]
            pltpu.sync_copy(out_v, out_hbm.at[pl.ds(base, CH)])

        stage(0, 0)

        def pair_body(g, carry):
            stage(2 * g + 1, 1)
            drain(0)
            compute(2 * g, 0)

            @pl.when(g < NCHUNK // 2 - 1)
            def _():
                stage(2 * g + 2, 0)

            drain(1)
            compute(2 * g + 1, 1)
            return carry

        lax.fori_loop(0, NCHUNK // 2, pair_body, 0)

    return k(proj, data, invb, bias16)


def kernel(data, length, table, W, b):
    data = data.astype(jnp.int32)
    lenf2d = length.astype(jnp.float32).reshape(BATCH // PACK, PACK)
    wt = jnp.zeros((EMB, PAD), jnp.float32).at[:, :2].set(W.T)
    wt_rep = jnp.tile(wt, (1, PACK))
    sel = (jnp.arange(NP) // PAD == jnp.arange(PACK)[:, None]
           ).astype(jnp.float32)
    bias16 = jnp.zeros((PAD,), jnp.float32).at[:2].set(b)
    packed = _project(table.T, wt_rep, sel)
    proj = packed.reshape(VOCAB, PAD)
    invb = _inv_packed(lenf2d, sel)
    out = _sc_pool(proj, data, invb, bias16)
    return out[:, :2]
